# baseline (speedup 1.0000x reference)
import functools

import jax
import jax.numpy as jnp
from jax import lax
from jax.experimental import pallas as pl
from jax.experimental.pallas import tpu as pltpu
from jax.experimental.pallas import tpu_sc as plsc

_TEMP_MIN = 2e-05
_NEG_INF = float("-inf")
_B, _V = 64, 100000
_NC, _NS, _L = 2, 16, 16
_NW = _NC * _NS
_ROWS_PER_W = _B // _NW
_NCH = _V // _L
_UNROLL = 10

_LN2 = 0.6931471805599453
_SQRT2 = 1.4142135381698608


def _allreduce(v, op):
    for sh in (8, 4, 2, 1):
        perm = lax.iota(jnp.int32, _L) ^ sh
        g = lax.gather(
            v, perm[:, None],
            lax.GatherDimensionNumbers(offset_dims=(),
                                       collapsed_slice_dims=(0,),
                                       start_index_map=(0,)),
            slice_sizes=(1,),
            mode=lax.GatherScatterMode.PROMISE_IN_BOUNDS)
        v = op(v, g)
    return v


def _logv(v):
    bits = lax.bitcast_convert_type(v, jnp.int32)
    ex = jnp.right_shift(bits, 23) & 255
    f = lax.bitcast_convert_type((bits & 0x007FFFFF) | 0x3F800000,
                                 jnp.float32)
    big = f > _SQRT2
    f = jnp.where(big, f * 0.5, f)
    k = (ex - 127 + jnp.where(big, 1, 0)).astype(jnp.float32)
    u = (f - 1.0) / (f + 1.0)
    w = u * u
    lf = u * (2.0 + w * (0.66666666666 + w * (0.4 + w * 0.2857142857)))
    return k * _LN2 + lf


def _body(x_hbm, p_hbm, out_hbm, samp_hbm, xv, pv, sv):
    wid = lax.axis_index("s") * _NC + lax.axis_index("c")

    for r2 in range(_ROWS_PER_W):
        row = wid * _ROWS_PER_W + r2
        pltpu.sync_copy(p_hbm.at[row], pv)
        tv = jnp.maximum(pv[pl.ds(0, _L)], _TEMP_MIN)
        rtv = 1.0 / tv
        minpv = pv[pl.ds(_L, _L)]
        epsv = pv[pl.ds(2 * _L, _L)]
        etav = pv[pl.ds(3 * _L, _L)]
        lminpv = jnp.where(minpv <= 1e-37, -1e30,
                           _logv(jnp.maximum(minpv, 1e-37)))

        pltpu.sync_copy(x_hbm.at[row], xv)

        @plsc.parallel_loop(0, _NCH, unroll=_UNROLL,
                            carry=jnp.full((_L,), _NEG_INF))
        def pa(i, acc):
            return jnp.maximum(acc, xv[pl.ds(i * _L, _L)])
        xmax = _allreduce(pa, jnp.maximum)

        @plsc.parallel_loop(0, _NCH, unroll=_UNROLL,
                            carry=(jnp.zeros((_L,), jnp.float32),
                                   jnp.full((_L,), _V, jnp.int32)))
        def pb(i, carry):
            z2a, topa = carry
            off = i * _L
            sm = (xv[pl.ds(off, _L)] - xmax) * rtv
            xv[pl.ds(off, _L)] = sm
            e = jnp.exp(sm)
            z2a = z2a + jnp.where(sm >= lminpv, e, 0.0)
            idx = lax.iota(jnp.int32, _L) + off
            topa = jnp.minimum(topa, jnp.where(e == 1.0, idx, _V))
            return z2a, topa
        z2a, topa = pb
        z2 = _allreduce(z2a, jnp.add)
        top = _allreduce(topa, jnp.minimum)

        lthr2 = jnp.maximum(lminpv, _logv(epsv * z2))

        @plsc.parallel_loop(0, _NCH, unroll=_UNROLL,
                            carry=(jnp.zeros((_L,), jnp.float32),
                                   jnp.zeros((_L,), jnp.float32)))
        def pc(i, carry):
            z3a, u3a = carry
            sm = xv[pl.ds(i * _L, _L)]
            z3c = jnp.where(sm >= lthr2, jnp.exp(sm), 0.0)
            return z3a + z3c, u3a + z3c * sm
        z3a, u3a = pc
        z3 = _allreduce(z3a, jnp.add) + jnp.where(lthr2 <= 0.0, 0.0, 1.0)
        u3 = _allreduce(u3a, jnp.add)

        neg_ent = u3 / z3 - _logv(z3)
        sqrt_eta = jnp.exp(0.5 * _logv(etav))
        eps_eta = jnp.minimum(etav, sqrt_eta * jnp.exp(neg_ent))
        lthr3 = jnp.maximum(lthr2, _logv(eps_eta * z3))

        @plsc.parallel_loop(0, _NCH, unroll=_UNROLL,
                            carry=jnp.zeros((_L,), jnp.float32))
        def pd(i, z4a):
            sm = xv[pl.ds(i * _L, _L)]
            return z4a + jnp.where(sm >= lthr3, jnp.exp(sm), 0.0)
        z4 = _allreduce(pd, jnp.add) + jnp.where(lthr3 <= 0.0, 0.0, 1.0)
        lz4 = _logv(z4)
        lthr3c = jnp.minimum(lthr3, 0.0)

        @plsc.parallel_loop(0, _NCH, unroll=_UNROLL, carry=jnp.int32(0))
        def pe(i, c):
            off = i * _L
            sm = xv[pl.ds(off, _L)]
            xv[pl.ds(off, _L)] = jnp.where(sm >= lthr3c, sm - lz4, _NEG_INF)
            return c
        del pe
        pltpu.sync_copy(xv, out_hbm.at[row])

        sv[...] = top
        pltpu.sync_copy(sv, samp_hbm.at[row])


def kernel(logits, temperature, min_p, epsilon_cutoff, eta_cutoff):
    B, V = logits.shape
    params = jnp.concatenate(
        [jnp.broadcast_to(p[:, None], (B, _L))
         for p in (temperature, min_p, epsilon_cutoff, eta_cutoff)], axis=1)
    mesh = plsc.VectorSubcoreMesh(core_axis_name="c", subcore_axis_name="s")
    f = functools.partial(
        pl.kernel, mesh=mesh,
        out_type=[jax.ShapeDtypeStruct((B, V), jnp.float32),
                  jax.ShapeDtypeStruct((B, _L), jnp.int32)],
        scratch_types=[pltpu.VMEM((V,), jnp.float32),
                       pltpu.VMEM((4 * _L,), jnp.float32),
                       pltpu.VMEM((_L,), jnp.int32)],
    )(_body)
    lp, samp = f(logits, params)
    return lp, samp[:, 0]


# speedup vs baseline: 1.7132x; 1.7132x over previous
import functools

import jax
import jax.numpy as jnp
from jax import lax
from jax.experimental import pallas as pl
from jax.experimental.pallas import tpu as pltpu
from jax.experimental.pallas import tpu_sc as plsc

_TEMP_MIN = 2e-05
_NEG_INF = float("-inf")
_B, _V = 64, 100000
_NC, _NS, _L = 2, 16, 16
_NW = _NC * _NS
_ROWS_PER_W = _B // _NW
_U = 25
_NITER = _V // (_L * _U)

_LN2 = 0.6931471805599453
_SQRT2 = 1.4142135381698608


def _allreduce(v, op):
    for sh in (8, 4, 2, 1):
        perm = lax.iota(jnp.int32, _L) ^ sh
        g = lax.gather(
            v, perm[:, None],
            lax.GatherDimensionNumbers(offset_dims=(),
                                       collapsed_slice_dims=(0,),
                                       start_index_map=(0,)),
            slice_sizes=(1,),
            mode=lax.GatherScatterMode.PROMISE_IN_BOUNDS)
        v = op(v, g)
    return v


def _logv(v):
    bits = lax.bitcast_convert_type(v, jnp.int32)
    ex = jnp.right_shift(bits, 23) & 255
    f = lax.bitcast_convert_type((bits & 0x007FFFFF) | 0x3F800000,
                                 jnp.float32)
    big = f > _SQRT2
    f = jnp.where(big, f * 0.5, f)
    k = (ex - 127 + jnp.where(big, 1, 0)).astype(jnp.float32)
    u = (f - 1.0) / (f + 1.0)
    w = u * u
    lf = u * (2.0 + w * (0.66666666666 + w * (0.4 + w * 0.2857142857)))
    return k * _LN2 + lf


def _body(x_hbm, p_hbm, out_hbm, samp_hbm, xv, pv, sv):
    wid = lax.axis_index("s") * _NC + lax.axis_index("c")

    for r2 in range(_ROWS_PER_W):
        row = wid * _ROWS_PER_W + r2
        pltpu.sync_copy(p_hbm.at[row], pv)
        tv = jnp.maximum(pv[pl.ds(0, _L)], _TEMP_MIN)
        rtv = 1.0 / tv
        minpv = pv[pl.ds(_L, _L)]
        epsv = pv[pl.ds(2 * _L, _L)]
        etav = pv[pl.ds(3 * _L, _L)]
        lminpv = jnp.where(minpv <= 1e-37, -1e30,
                           _logv(jnp.maximum(minpv, 1e-37)))

        pltpu.sync_copy(x_hbm.at[row], xv)

        def pa(i, acc):
            for j in range(_U):
                acc = jnp.maximum(acc, xv[pl.ds((i * _U + j) * _L, _L)])
            return acc
        xmax = _allreduce(lax.fori_loop(0, _NITER, pa,
                                        jnp.full((_L,), _NEG_INF)),
                          jnp.maximum)

        def pb(i, carry):
            z2a, z2b, topa, topb = carry
            for j in range(_U):
                off = (i * _U + j) * _L
                sm = (xv[pl.ds(off, _L)] - xmax) * rtv
                xv[pl.ds(off, _L)] = sm
                e = jnp.exp(sm)
                zc = jnp.where(sm >= lminpv, e, 0.0)
                idx = lax.iota(jnp.int32, _L) + off
                tc = jnp.where(e == 1.0, idx, _V)
                if j % 2 == 0:
                    z2a = z2a + zc
                    topa = jnp.minimum(topa, tc)
                else:
                    z2b = z2b + zc
                    topb = jnp.minimum(topb, tc)
            return z2a, z2b, topa, topb
        z2a, z2b, topa, topb = lax.fori_loop(
            0, _NITER, pb,
            (jnp.zeros((_L,), jnp.float32), jnp.zeros((_L,), jnp.float32),
             jnp.full((_L,), _V, jnp.int32), jnp.full((_L,), _V, jnp.int32)))
        z2 = _allreduce(z2a + z2b, jnp.add)
        top = _allreduce(jnp.minimum(topa, topb), jnp.minimum)

        lthr2 = jnp.maximum(lminpv, _logv(epsv * z2))

        def pc(i, carry):
            z3a, z3b, u3a, u3b = carry
            for j in range(_U):
                sm = xv[pl.ds((i * _U + j) * _L, _L)]
                z3c = jnp.where(sm >= lthr2, jnp.exp(sm), 0.0)
                if j % 2 == 0:
                    z3a = z3a + z3c
                    u3a = u3a + z3c * sm
                else:
                    z3b = z3b + z3c
                    u3b = u3b + z3c * sm
            return z3a, z3b, u3a, u3b
        z3a, z3b, u3a, u3b = lax.fori_loop(
            0, _NITER, pc,
            (jnp.zeros((_L,), jnp.float32), jnp.zeros((_L,), jnp.float32),
             jnp.zeros((_L,), jnp.float32), jnp.zeros((_L,), jnp.float32)))
        z3 = _allreduce(z3a + z3b, jnp.add) + jnp.where(lthr2 <= 0.0, 0.0, 1.0)
        u3 = _allreduce(u3a + u3b, jnp.add)

        neg_ent = u3 / z3 - _logv(z3)
        sqrt_eta = jnp.exp(0.5 * _logv(etav))
        eps_eta = jnp.minimum(etav, sqrt_eta * jnp.exp(neg_ent))
        lthr3 = jnp.maximum(lthr2, _logv(eps_eta * z3))

        def pd(i, carry):
            z4a, z4b = carry
            for j in range(_U):
                sm = xv[pl.ds((i * _U + j) * _L, _L)]
                zc = jnp.where(sm >= lthr3, jnp.exp(sm), 0.0)
                if j % 2 == 0:
                    z4a = z4a + zc
                else:
                    z4b = z4b + zc
            return z4a, z4b
        z4a, z4b = lax.fori_loop(
            0, _NITER, pd,
            (jnp.zeros((_L,), jnp.float32), jnp.zeros((_L,), jnp.float32)))
        z4 = _allreduce(z4a + z4b, jnp.add) + jnp.where(lthr3 <= 0.0, 0.0, 1.0)
        lz4 = _logv(z4)
        lthr3c = jnp.minimum(lthr3, 0.0)

        def pe(i, c):
            for j in range(_U):
                off = (i * _U + j) * _L
                sm = xv[pl.ds(off, _L)]
                xv[pl.ds(off, _L)] = jnp.where(sm >= lthr3c, sm - lz4,
                                               _NEG_INF)
            return c
        lax.fori_loop(0, _NITER, pe, jnp.int32(0))
        pltpu.sync_copy(xv, out_hbm.at[row])

        sv[...] = top
        pltpu.sync_copy(sv, samp_hbm.at[row])


def kernel(logits, temperature, min_p, epsilon_cutoff, eta_cutoff):
    B, V = logits.shape
    params = jnp.concatenate(
        [jnp.broadcast_to(p[:, None], (B, _L))
         for p in (temperature, min_p, epsilon_cutoff, eta_cutoff)], axis=1)
    mesh = plsc.VectorSubcoreMesh(core_axis_name="c", subcore_axis_name="s")
    f = functools.partial(
        pl.kernel, mesh=mesh,
        out_type=[jax.ShapeDtypeStruct((B, V), jnp.float32),
                  jax.ShapeDtypeStruct((B, _L), jnp.int32)],
        scratch_types=[pltpu.VMEM((V,), jnp.float32),
                       pltpu.VMEM((4 * _L,), jnp.float32),
                       pltpu.VMEM((_L,), jnp.int32)],
    )(_body)
    lp, samp = f(logits, params)
    return lp, samp[:, 0]


# speedup vs baseline: 7.6493x; 4.4648x over previous
import functools

import jax
import jax.numpy as jnp
from jax.experimental import pallas as pl
from jax.experimental.pallas import tpu as pltpu

_TEMP_MIN = 2e-05
_NEG_INF = float("-inf")
_W = 1024


def _body(t_ref, minp_ref, eps_ref, eta_ref, x_ref, out_ref, samp_ref,
          sm_ref, e_ref):
    R, V = x_ref.shape
    NF = V // _W
    TW = V - NF * _W
    toff = NF * _W

    rt = 1.0 / jnp.maximum(t_ref[...], _TEMP_MIN)
    lminp = jnp.log(minp_ref[...])

    acc = x_ref[:, pl.ds(0, _W)]
    for i in range(1, NF):
        acc = jnp.maximum(acc, x_ref[:, pl.ds(i * _W, _W)])
    xmax = jnp.max(acc, axis=-1, keepdims=True)
    xmax = jnp.maximum(xmax, jnp.max(x_ref[:, pl.ds(toff, TW)],
                                     axis=-1, keepdims=True))

    def p2_chunk(off, w):
        x = x_ref[:, pl.ds(off, w)]
        sm = (x - xmax) * rt
        e = jnp.exp(sm)
        sm_ref[:, pl.ds(off, w)] = sm
        e_ref[:, pl.ds(off, w)] = e
        z2c = jnp.where(sm >= lminp, e, 0.0)
        idx = jax.lax.broadcasted_iota(jnp.int32, (R, w), 1) + off
        topc = jnp.where(e == 1.0, idx, V)
        return z2c, topc

    z2a, topa = p2_chunk(0, _W)
    for i in range(1, NF):
        z2c, topc = p2_chunk(i * _W, _W)
        z2a = z2a + z2c
        topa = jnp.minimum(topa, topc)
    z2tc, toptc = p2_chunk(toff, TW)
    z2 = (jnp.sum(z2a, axis=-1, keepdims=True)
          + jnp.sum(z2tc, axis=-1, keepdims=True))
    top_idx = jnp.minimum(jnp.min(topa, axis=-1, keepdims=True),
                          jnp.min(toptc, axis=-1, keepdims=True))

    lthr2 = jnp.maximum(lminp, jnp.log(eps_ref[...] * z2))

    def p3_chunk(off, w):
        sm = sm_ref[:, pl.ds(off, w)]
        e = e_ref[:, pl.ds(off, w)]
        z3c = jnp.where(sm >= lthr2, e, 0.0)
        return z3c, z3c * sm

    z3a, u3a = p3_chunk(0, _W)
    for i in range(1, NF):
        z3c, u3c = p3_chunk(i * _W, _W)
        z3a = z3a + z3c
        u3a = u3a + u3c
    z3tc, u3tc = p3_chunk(toff, TW)
    z3 = (jnp.sum(z3a, axis=-1, keepdims=True)
          + jnp.sum(z3tc, axis=-1, keepdims=True))
    u3 = (jnp.sum(u3a, axis=-1, keepdims=True)
          + jnp.sum(u3tc, axis=-1, keepdims=True))
    z3 = z3 + jnp.where(lthr2 <= 0.0, 0.0, 1.0)

    neg_ent = u3 / z3 - jnp.log(z3)
    eta = eta_ref[...]
    eps_eta = jnp.minimum(eta, jnp.sqrt(eta) * jnp.exp(neg_ent))
    lthr3 = jnp.maximum(lthr2, jnp.log(eps_eta * z3))

    def p4_chunk(off, w):
        sm = sm_ref[:, pl.ds(off, w)]
        e = e_ref[:, pl.ds(off, w)]
        return jnp.where(sm >= lthr3, e, 0.0)

    z4a = p4_chunk(0, _W)
    for i in range(1, NF):
        z4a = z4a + p4_chunk(i * _W, _W)
    z4 = (jnp.sum(z4a, axis=-1, keepdims=True)
          + jnp.sum(p4_chunk(toff, TW), axis=-1, keepdims=True))
    z4 = z4 + jnp.where(lthr3 <= 0.0, 0.0, 1.0)
    lz4 = jnp.log(z4)

    lthr3c = jnp.minimum(lthr3, 0.0)

    def p5_chunk(off, w):
        sm = sm_ref[:, pl.ds(off, w)]
        out_ref[:, pl.ds(off, w)] = jnp.where(sm >= lthr3c, sm - lz4, _NEG_INF)

    for i in range(NF):
        p5_chunk(i * _W, _W)
    p5_chunk(toff, TW)

    samp_ref[...] = top_idx


def kernel(logits, temperature, min_p, epsilon_cutoff, eta_cutoff):
    B, V = logits.shape
    R = 8
    grid = (B // R,)
    row_spec = pl.BlockSpec((R, 1), lambda i: (i, 0))
    out = pl.pallas_call(
        _body,
        grid=grid,
        in_specs=[row_spec, row_spec, row_spec, row_spec,
                  pl.BlockSpec((R, V), lambda i: (i, 0))],
        out_specs=[pl.BlockSpec((R, V), lambda i: (i, 0)),
                   pl.BlockSpec((R, 1), lambda i: (i, 0))],
        out_shape=[jax.ShapeDtypeStruct((B, V), jnp.float32),
                   jax.ShapeDtypeStruct((B, 1), jnp.int32)],
        scratch_shapes=[pltpu.VMEM((R, V), jnp.float32),
                        pltpu.VMEM((R, V), jnp.float32)],
        compiler_params=pltpu.CompilerParams(
            dimension_semantics=("parallel",)),
    )(temperature.reshape(B, 1), min_p.reshape(B, 1),
      epsilon_cutoff.reshape(B, 1), eta_cutoff.reshape(B, 1), logits)
    return out[0], out[1].reshape(B)
